# baseline (speedup 1.0000x reference)
import functools

import jax
import jax.numpy as jnp
from jax import lax
from jax.experimental import pallas as pl
from jax.experimental.pallas import tpu as pltpu
from jax.experimental.pallas import tpu_sc as plsc

E = 320000
KEY_DIM = 128
OUT_DIM = 128
S = 10000
QDIM = 80
S_PAD = QDIM * 128
B_EDGE = 2560
N_CORES = 2
N_SUB = 16
N_TILES = N_CORES * N_SUB
PER_TILE = E // N_TILES
CH = 80
N_CHUNK = PER_TILE // CH
ROWS_PER_SUB = S_PAD // N_SUB


def _emb_body(keys_ref, vals_ref, qcol_ref, rcol_ref, embW_ref, embb_ref,
              swk_ref, swq_ref, q_ref, sb_ref, y_ref, d_ref):
    i = pl.program_id(0)
    c = jnp.sum(q_ref[...] * swq_ref[...]) + sb_ref[0, 0]
    scores = jnp.dot(keys_ref[...], swk_ref[...],
                     preferred_element_type=jnp.float32) + c
    p = jnp.exp(scores)
    emb = jnp.dot(vals_ref[...], embW_ref[...],
                  preferred_element_type=jnp.float32) + embb_ref[...]
    y_ref[...] = jnp.maximum(emb, 0.0) * p
    qf = qcol_ref[...].astype(jnp.float32)
    rf = rcol_ref[...].astype(jnp.float32)
    iq = lax.broadcasted_iota(jnp.int32, (1, QDIM), 1).astype(jnp.float32)
    ir = lax.broadcasted_iota(jnp.int32, (1, 128), 1).astype(jnp.float32)
    qh = jnp.where(qf == iq, p, 0.0)
    rh = jnp.where(rf == ir, 1.0, 0.0)
    contrib = lax.dot_general(qh, rh, (((0,), (0,)), ((), ())),
                              preferred_element_type=jnp.float32)

    @pl.when(i == 0)
    def _():
        d_ref[...] = jnp.zeros_like(d_ref)

    d_ref[...] += contrib


def _make_rows(keys, values, qcol, rcol, emb_W, emb_b, score_W, score_b,
               query):
    swk = score_W[:KEY_DIM]
    swq = score_W[KEY_DIM:, 0][None, :]
    grid = E // B_EDGE
    return pl.pallas_call(
        _emb_body,
        grid=(grid,),
        in_specs=[
            pl.BlockSpec((B_EDGE, KEY_DIM), lambda i: (i, 0)),
            pl.BlockSpec((B_EDGE, KEY_DIM), lambda i: (i, 0)),
            pl.BlockSpec((B_EDGE, 1), lambda i: (i, 0)),
            pl.BlockSpec((B_EDGE, 1), lambda i: (i, 0)),
            pl.BlockSpec((KEY_DIM, OUT_DIM), lambda i: (0, 0)),
            pl.BlockSpec((1, OUT_DIM), lambda i: (0, 0)),
            pl.BlockSpec((KEY_DIM, 1), lambda i: (0, 0)),
            pl.BlockSpec((1, swq.shape[1]), lambda i: (0, 0)),
            pl.BlockSpec((1, swq.shape[1]), lambda i: (0, 0)),
            pl.BlockSpec((1, 1), lambda i: (0, 0)),
        ],
        out_specs=[
            pl.BlockSpec((B_EDGE, OUT_DIM), lambda i: (i, 0)),
            pl.BlockSpec((QDIM, 128), lambda i: (0, 0)),
        ],
        out_shape=[
            jax.ShapeDtypeStruct((E, OUT_DIM), jnp.float32),
            jax.ShapeDtypeStruct((QDIM, 128), jnp.float32),
        ],
    )(keys, values, qcol, rcol, emb_W, emb_b[None, :], swk, swq,
      query[None, :], score_b.reshape(1, 1))


NBUF = 4


def _sc_body(y_hbm, idx_hbm, zero_hbm, out_hbm, i0, i1, i2, i3, bufs,
             acc_s, *sems):
    c = lax.axis_index("c")
    s = lax.axis_index("s")
    wid = c * N_SUB + s
    base = wid * PER_TILE
    idx_bufs = (i0, i1, i2, i3)
    pltpu.sync_copy(zero_hbm.at[pl.ds(s * ROWS_PER_SUB, ROWS_PER_SUB)],
                    acc_s.at[pl.ds(s * ROWS_PER_SUB, ROWS_PER_SUB)])
    plsc.subcore_barrier()

    def gather_rows(j, b):
        return pltpu.make_async_copy(
            y_hbm.at[pl.ds(base + j * CH, CH)], bufs.at[b], sems[b])

    def gather_idx(j, b):
        return pltpu.make_async_copy(
            idx_hbm.at[pl.ds(base + j * CH, CH)], idx_bufs[b], sems[NBUF + b])

    for b in range(NBUF):
        gather_rows(b, b).start()
        gather_idx(b, b).start()

    def step(jj, carry):
        for b in range(NBUF):
            j = jj * NBUF + b
            gather_rows(j, b).wait()
            gather_idx(j, b).wait()
            pltpu.sync_copy(bufs.at[b], acc_s.at[idx_bufs[b]], add=True)

            @pl.when(j + NBUF < N_CHUNK)
            def _():
                gather_rows(j + NBUF, b).start()
                gather_idx(j + NBUF, b).start()

        return carry

    lax.fori_loop(0, N_CHUNK // NBUF, step, 0)
    jt = (N_CHUNK // NBUF) * NBUF
    gather_rows(jt, 0).wait()
    gather_idx(jt, 0).wait()
    pltpu.sync_copy(bufs.at[0], acc_s.at[idx_bufs[0]], add=True)

    plsc.subcore_barrier()
    pltpu.sync_copy(acc_s.at[pl.ds(s * ROWS_PER_SUB, ROWS_PER_SUB)],
                    out_hbm.at[c, pl.ds(s * ROWS_PER_SUB, ROWS_PER_SUB)])


def _sc_aggregate(rows, idx32, zeros_acc):
    mesh = plsc.VectorSubcoreMesh(core_axis_name="c", subcore_axis_name="s")
    k = functools.partial(
        pl.kernel,
        mesh=mesh,
        out_type=jax.ShapeDtypeStruct((N_CORES, S_PAD, OUT_DIM), jnp.float32),
        scratch_types=[
            pltpu.VMEM((CH,), jnp.int32),
            pltpu.VMEM((CH,), jnp.int32),
            pltpu.VMEM((CH,), jnp.int32),
            pltpu.VMEM((CH,), jnp.int32),
            pltpu.VMEM((NBUF, CH, OUT_DIM), jnp.float32),
            pltpu.VMEM_SHARED((S_PAD, OUT_DIM), jnp.float32),
        ] + [pltpu.SemaphoreType.DMA] * (2 * NBUF),
    )(_sc_body)
    return k(rows, idx32, zeros_acc)


def _combine_body(acc_ref, den_ref, out_ref):
    a = acc_ref[0] + acc_ref[1]
    out_ref[...] = a[:S] / (den_ref[...] + 1e-16)


def _combine(partials, den_col):
    return pl.pallas_call(
        _combine_body,
        out_shape=jax.ShapeDtypeStruct((S, OUT_DIM), jnp.float32),
    )(partials, den_col)


def kernel(keys, values, query, index, size, emb_W, emb_b, score_W, score_b):
    idx32 = index.astype(jnp.int32)
    qcol = (idx32 // 128)[:, None]
    rcol = (idx32 % 128)[:, None]
    rows, dmat = _make_rows(keys, values, qcol, rcol, emb_W, emb_b,
                            score_W, score_b, query)
    zeros_acc = jnp.zeros((S_PAD, OUT_DIM), jnp.float32)
    partials = _sc_aggregate(rows, idx32, zeros_acc)
    den_col = dmat.reshape(-1)[:S, None]
    return _combine(partials, den_col)


# speedup vs baseline: 2.0055x; 2.0055x over previous
import functools

import jax
import jax.numpy as jnp
from jax import lax
from jax.experimental import pallas as pl
from jax.experimental.pallas import tpu as pltpu
from jax.experimental.pallas import tpu_sc as plsc

E = 320000
KEY_DIM = 128
OUT_DIM = 128
S = 10000
QDIM = 80
S_PAD = QDIM * 128
B_EDGE = 2560
N_CORES = 2
N_SUB = 16
N_TILES = N_CORES * N_SUB
PER_TILE = E // N_TILES
CH = 80
N_CHUNK = PER_TILE // CH
ROWS_PER_SUB = S_PAD // N_SUB


def _emb_body(keys_ref, vals_ref, idx_ref, embW_ref, embb_ref,
              swk_ref, swq_ref, q_ref, sb_ref, y_ref, d_ref):
    i = pl.program_id(0)
    c = jnp.sum(q_ref[...] * swq_ref[...]) + sb_ref[0, 0]
    keys = keys_ref[...]
    scores = jnp.dot(keys, swk_ref[...],
                     preferred_element_type=jnp.float32) + c
    p = jnp.exp(scores)
    emb = jnp.dot(vals_ref[...], embW_ref[...],
                  preferred_element_type=jnp.float32) + embb_ref[...]
    y_ref[...] = jnp.maximum(emb, 0.0) * p
    scores_r = lax.dot_general(swk_ref[...].reshape(1, KEY_DIM), keys,
                               (((1,), (1,)), ((), ())),
                               preferred_element_type=jnp.float32) + c
    p_r = jnp.exp(scores_r)
    idx = idx_ref[0]
    qf = (idx // 128).astype(jnp.float32)
    rf = (idx % 128).astype(jnp.float32)
    iq = lax.broadcasted_iota(jnp.int32, (QDIM, 1), 0).astype(jnp.float32)
    ir = lax.broadcasted_iota(jnp.int32, (128, 1), 0).astype(jnp.float32)
    qh = jnp.where(qf == iq, p_r, 0.0)
    rh = jnp.where(rf == ir, 1.0, 0.0)
    contrib = lax.dot_general(qh, rh, (((1,), (1,)), ((), ())),
                              preferred_element_type=jnp.float32)

    @pl.when(i == 0)
    def _():
        d_ref[...] = jnp.zeros_like(d_ref)

    d_ref[...] += contrib


def _make_rows(keys, values, idxd, emb_W, emb_b, score_W, score_b, query):
    swk = score_W[:KEY_DIM]
    swq = score_W[KEY_DIM:, 0][None, :]
    grid = E // B_EDGE
    return pl.pallas_call(
        _emb_body,
        grid=(grid,),
        in_specs=[
            pl.BlockSpec((B_EDGE, KEY_DIM), lambda i: (i, 0)),
            pl.BlockSpec((B_EDGE, KEY_DIM), lambda i: (i, 0)),
            pl.BlockSpec((1, 1, B_EDGE), lambda i: (i, 0, 0)),
            pl.BlockSpec((KEY_DIM, OUT_DIM), lambda i: (0, 0)),
            pl.BlockSpec((1, OUT_DIM), lambda i: (0, 0)),
            pl.BlockSpec((KEY_DIM, 1), lambda i: (0, 0)),
            pl.BlockSpec((1, swq.shape[1]), lambda i: (0, 0)),
            pl.BlockSpec((1, swq.shape[1]), lambda i: (0, 0)),
            pl.BlockSpec((1, 1), lambda i: (0, 0)),
        ],
        out_specs=[
            pl.BlockSpec((B_EDGE, OUT_DIM), lambda i: (i, 0)),
            pl.BlockSpec((QDIM, 128), lambda i: (0, 0)),
        ],
        out_shape=[
            jax.ShapeDtypeStruct((E, OUT_DIM), jnp.float32),
            jax.ShapeDtypeStruct((QDIM, 128), jnp.float32),
        ],
    )(keys, values, idxd, emb_W, emb_b[None, :], swk, swq,
      query[None, :], score_b.reshape(1, 1))


NBUF = 4


def _sc_body(y_hbm, idx_hbm, zero_hbm, out_hbm, i0, i1, i2, i3, bufs,
             acc_s, *sems):
    c = lax.axis_index("c")
    s = lax.axis_index("s")
    wid = c * N_SUB + s
    base = wid * PER_TILE
    idx_bufs = (i0, i1, i2, i3)
    pltpu.sync_copy(zero_hbm.at[pl.ds(s * ROWS_PER_SUB, ROWS_PER_SUB)],
                    acc_s.at[pl.ds(s * ROWS_PER_SUB, ROWS_PER_SUB)])
    plsc.subcore_barrier()

    def gather_rows(j, b):
        return pltpu.make_async_copy(
            y_hbm.at[pl.ds(base + j * CH, CH)], bufs.at[b], sems[b])

    def gather_idx(j, b):
        return pltpu.make_async_copy(
            idx_hbm.at[pl.ds(base + j * CH, CH)], idx_bufs[b], sems[NBUF + b])

    for b in range(NBUF):
        gather_rows(b, b).start()
        gather_idx(b, b).start()

    def step(jj, carry):
        for b in range(NBUF):
            j = jj * NBUF + b
            gather_rows(j, b).wait()
            gather_idx(j, b).wait()
            pltpu.sync_copy(bufs.at[b], acc_s.at[idx_bufs[b]], add=True)

            @pl.when(j + NBUF < N_CHUNK)
            def _():
                gather_rows(j + NBUF, b).start()
                gather_idx(j + NBUF, b).start()

        return carry

    lax.fori_loop(0, N_CHUNK // NBUF, step, 0)
    jt = (N_CHUNK // NBUF) * NBUF
    gather_rows(jt, 0).wait()
    gather_idx(jt, 0).wait()
    pltpu.sync_copy(bufs.at[0], acc_s.at[idx_bufs[0]], add=True)

    plsc.subcore_barrier()
    pltpu.sync_copy(acc_s.at[pl.ds(s * ROWS_PER_SUB, ROWS_PER_SUB)],
                    out_hbm.at[c, pl.ds(s * ROWS_PER_SUB, ROWS_PER_SUB)])


def _sc_aggregate(rows, idx32, zeros_acc):
    mesh = plsc.VectorSubcoreMesh(core_axis_name="c", subcore_axis_name="s")
    k = functools.partial(
        pl.kernel,
        mesh=mesh,
        out_type=jax.ShapeDtypeStruct((N_CORES, S_PAD, OUT_DIM), jnp.float32),
        scratch_types=[
            pltpu.VMEM((CH,), jnp.int32),
            pltpu.VMEM((CH,), jnp.int32),
            pltpu.VMEM((CH,), jnp.int32),
            pltpu.VMEM((CH,), jnp.int32),
            pltpu.VMEM((NBUF, CH, OUT_DIM), jnp.float32),
            pltpu.VMEM_SHARED((S_PAD, OUT_DIM), jnp.float32),
        ] + [pltpu.SemaphoreType.DMA] * (2 * NBUF),
    )(_sc_body)
    return k(rows, idx32, zeros_acc)


def _combine_body(acc_ref, den_ref, out_ref):
    a = acc_ref[0] + acc_ref[1]
    out_ref[...] = a[:S] / (den_ref[...] + 1e-16)


def _combine(partials, den_col):
    return pl.pallas_call(
        _combine_body,
        out_shape=jax.ShapeDtypeStruct((S, OUT_DIM), jnp.float32),
    )(partials, den_col)


def kernel(keys, values, query, index, size, emb_W, emb_b, score_W, score_b):
    idx32 = index.astype(jnp.int32)
    idxd = idx32.reshape(E // B_EDGE, 1, B_EDGE)
    rows, dmat = _make_rows(keys, values, idxd, emb_W, emb_b,
                            score_W, score_b, query)
    zeros_acc = jnp.zeros((S_PAD, OUT_DIM), jnp.float32)
    partials = _sc_aggregate(rows, idx32, zeros_acc)
    den_col = dmat.reshape(-1)[:S, None]
    return _combine(partials, den_col)


# speedup vs baseline: 2.1069x; 1.0505x over previous
import functools

import jax
import jax.numpy as jnp
from jax import lax
from jax.experimental import pallas as pl
from jax.experimental.pallas import tpu as pltpu
from jax.experimental.pallas import tpu_sc as plsc

E = 320000
KEY_DIM = 128
OUT_DIM = 128
S = 10000
QDIM = 80
S_PAD = QDIM * 128
B_EDGE = 2560
N_BLOCKS = E // B_EDGE
N_BLK_A = 62
N_CORES = 2
N_SUB = 16
N_TILES = N_CORES * N_SUB
CH = 80
ROWS_PER_SUB = S_PAD // N_SUB
NBUF = 4


def _emb_body(keys_ref, vals_ref, idx_ref, embW_ref, embb_ref,
              swk_ref, swq_ref, q_ref, sb_ref, y_ref, d_ref):
    i = pl.program_id(0)
    c = jnp.sum(q_ref[...] * swq_ref[...]) + sb_ref[0, 0]
    keys = keys_ref[...]
    scores = jnp.dot(keys, swk_ref[...],
                     preferred_element_type=jnp.float32) + c
    p = jnp.exp(scores)
    emb = jnp.dot(vals_ref[...], embW_ref[...],
                  preferred_element_type=jnp.float32) + embb_ref[...]
    y_ref[...] = jnp.maximum(emb, 0.0) * p
    scores_r = lax.dot_general(swk_ref[...].reshape(1, KEY_DIM), keys,
                               (((1,), (1,)), ((), ())),
                               preferred_element_type=jnp.float32) + c
    p_r = jnp.exp(scores_r)
    idx = idx_ref[0]
    qf = (idx // 128).astype(jnp.float32)
    rf = (idx % 128).astype(jnp.float32)
    iq = lax.broadcasted_iota(jnp.int32, (QDIM, 1), 0).astype(jnp.float32)
    ir = lax.broadcasted_iota(jnp.int32, (128, 1), 0).astype(jnp.float32)
    qh = jnp.where(qf == iq, p_r, 0.0)
    rh = jnp.where(rf == ir, 1.0, 0.0)
    contrib = lax.dot_general(qh, rh, (((1,), (1,)), ((), ())),
                              preferred_element_type=jnp.float32)

    @pl.when(i == 0)
    def _():
        d_ref[...] = jnp.zeros_like(d_ref)

    d_ref[...] += contrib


def _make_rows(keys, values, idxd, blk0, nblk, emb_W, emb_b, score_W,
               score_b, query):
    swk = score_W[:KEY_DIM]
    swq = score_W[KEY_DIM:, 0][None, :]
    n_edges = nblk * B_EDGE
    grid = nblk
    return pl.pallas_call(
        _emb_body,
        grid=(grid,),
        in_specs=[
            pl.BlockSpec((B_EDGE, KEY_DIM), lambda i: (i + blk0, 0)),
            pl.BlockSpec((B_EDGE, KEY_DIM), lambda i: (i + blk0, 0)),
            pl.BlockSpec((1, 1, B_EDGE), lambda i: (i + blk0, 0, 0)),
            pl.BlockSpec((KEY_DIM, OUT_DIM), lambda i: (0, 0)),
            pl.BlockSpec((1, OUT_DIM), lambda i: (0, 0)),
            pl.BlockSpec((KEY_DIM, 1), lambda i: (0, 0)),
            pl.BlockSpec((1, swq.shape[1]), lambda i: (0, 0)),
            pl.BlockSpec((1, swq.shape[1]), lambda i: (0, 0)),
            pl.BlockSpec((1, 1), lambda i: (0, 0)),
        ],
        out_specs=[
            pl.BlockSpec((B_EDGE, OUT_DIM), lambda i: (i, 0)),
            pl.BlockSpec((QDIM, 128), lambda i: (0, 0)),
        ],
        out_shape=[
            jax.ShapeDtypeStruct((n_edges, OUT_DIM), jnp.float32),
            jax.ShapeDtypeStruct((QDIM, 128), jnp.float32),
        ],
    )(keys, values, idxd, emb_W, emb_b[None, :], swk, swq,
      query[None, :], score_b.reshape(1, 1))


def _sc_body(n_chunk, e0, y_hbm, idx_hbm, zero_hbm, out_hbm, i0, i1, i2, i3,
             bufs, acc_s, *sems):
    c = lax.axis_index("c")
    s = lax.axis_index("s")
    wid = c * N_SUB + s
    base = wid * (n_chunk * CH)
    ibase = e0 + base
    idx_bufs = (i0, i1, i2, i3)
    pltpu.sync_copy(zero_hbm.at[pl.ds(s * ROWS_PER_SUB, ROWS_PER_SUB)],
                    acc_s.at[pl.ds(s * ROWS_PER_SUB, ROWS_PER_SUB)])
    plsc.subcore_barrier()

    def gather_rows(j, b):
        return pltpu.make_async_copy(
            y_hbm.at[pl.ds(base + j * CH, CH)], bufs.at[b], sems[b])

    def gather_idx(j, b):
        return pltpu.make_async_copy(
            idx_hbm.at[pl.ds(ibase + j * CH, CH)], idx_bufs[b],
            sems[NBUF + b])

    def consume(j, b):
        gather_rows(j, b).wait()
        gather_idx(j, b).wait()
        pltpu.sync_copy(bufs.at[b], acc_s.at[idx_bufs[b]], add=True)

    for b in range(NBUF):
        gather_rows(b, b).start()
        gather_idx(b, b).start()

    def step(jj, carry):
        for b in range(NBUF):
            j = jj * NBUF + b
            consume(j, b)

            @pl.when(j + NBUF < n_chunk)
            def _():
                gather_rows(j + NBUF, b).start()
                gather_idx(j + NBUF, b).start()

        return carry

    lax.fori_loop(0, n_chunk // NBUF, step, 0)
    for b in range(n_chunk % NBUF):
        consume((n_chunk // NBUF) * NBUF + b, b)

    plsc.subcore_barrier()
    pltpu.sync_copy(acc_s.at[pl.ds(s * ROWS_PER_SUB, ROWS_PER_SUB)],
                    out_hbm.at[c, pl.ds(s * ROWS_PER_SUB, ROWS_PER_SUB)])


def _sc_aggregate(rows, idx32, zeros_acc, e0):
    n_chunk = rows.shape[0] // (N_TILES * CH)
    mesh = plsc.VectorSubcoreMesh(core_axis_name="c", subcore_axis_name="s")
    k = functools.partial(
        pl.kernel,
        mesh=mesh,
        out_type=jax.ShapeDtypeStruct((N_CORES, S_PAD, OUT_DIM), jnp.float32),
        scratch_types=[
            pltpu.VMEM((CH,), jnp.int32),
            pltpu.VMEM((CH,), jnp.int32),
            pltpu.VMEM((CH,), jnp.int32),
            pltpu.VMEM((CH,), jnp.int32),
            pltpu.VMEM((NBUF, CH, OUT_DIM), jnp.float32),
            pltpu.VMEM_SHARED((S_PAD, OUT_DIM), jnp.float32),
        ] + [pltpu.SemaphoreType.DMA] * (2 * NBUF),
    )(functools.partial(_sc_body, n_chunk, e0))
    return k(rows, idx32, zeros_acc)


def _combine_body(pa_ref, pb_ref, da_ref, db_ref, out_ref):
    a = (pa_ref[0] + pa_ref[1]) + (pb_ref[0] + pb_ref[1])
    den = da_ref[...] + db_ref[...] + 1e-16
    out_ref[...] = a[:S] / den


def _combine(pa, pb, da_col, db_col):
    return pl.pallas_call(
        _combine_body,
        out_shape=jax.ShapeDtypeStruct((S, OUT_DIM), jnp.float32),
    )(pa, pb, da_col, db_col)


def kernel(keys, values, query, index, size, emb_W, emb_b, score_W, score_b):
    idx32 = index.astype(jnp.int32)
    idxd = idx32.reshape(N_BLOCKS, 1, B_EDGE)
    ea = N_BLK_A * B_EDGE
    rows_a, dmat_a = _make_rows(keys, values, idxd, 0, N_BLK_A,
                                emb_W, emb_b, score_W, score_b, query)
    rows_b, dmat_b = _make_rows(keys, values, idxd, N_BLK_A,
                                N_BLOCKS - N_BLK_A,
                                emb_W, emb_b, score_W, score_b, query)
    zeros_acc = jnp.zeros((S_PAD, OUT_DIM), jnp.float32)
    part_a = _sc_aggregate(rows_a, idx32, zeros_acc, 0)
    part_b = _sc_aggregate(rows_b, idx32, zeros_acc, ea)
    da_col = dmat_a.reshape(-1)[:S, None]
    db_col = dmat_b.reshape(-1)[:S, None]
    return _combine(part_a, part_b, da_col, db_col)


# speedup vs baseline: 2.1690x; 1.0295x over previous
import functools

import jax
import jax.numpy as jnp
from jax import lax
from jax.experimental import pallas as pl
from jax.experimental.pallas import tpu as pltpu
from jax.experimental.pallas import tpu_sc as plsc

E = 320000
KEY_DIM = 128
OUT_DIM = 128
S = 10000
QDIM = 80
S_PAD = QDIM * 128
B_EDGE = 2560
N_BLOCKS = E // B_EDGE
SPLITS = (31, 31, 31, 32)
N_CORES = 2
N_SUB = 16
N_TILES = N_CORES * N_SUB
CH = 80
ROWS_PER_SUB = S_PAD // N_SUB
NBUF = 4


def _emb_body(keys_ref, vals_ref, idx_ref, dprev_ref, embW_ref, embb_ref,
              swk_ref, swq_ref, q_ref, sb_ref, y_ref, d_ref):
    i = pl.program_id(0)
    c = jnp.sum(q_ref[...] * swq_ref[...]) + sb_ref[0, 0]
    keys = keys_ref[...]
    scores = jnp.dot(keys, swk_ref[...],
                     preferred_element_type=jnp.float32) + c
    p = jnp.exp(scores)
    emb = jnp.dot(vals_ref[...], embW_ref[...],
                  preferred_element_type=jnp.float32) + embb_ref[...]
    y_ref[...] = jnp.maximum(emb, 0.0) * p
    scores_r = lax.dot_general(swk_ref[...].reshape(1, KEY_DIM), keys,
                               (((1,), (1,)), ((), ())),
                               preferred_element_type=jnp.float32) + c
    p_r = jnp.exp(scores_r)
    idx = idx_ref[0]
    qf = (idx // 128).astype(jnp.float32)
    rf = (idx % 128).astype(jnp.float32)
    iq = lax.broadcasted_iota(jnp.int32, (QDIM, 1), 0).astype(jnp.float32)
    ir = lax.broadcasted_iota(jnp.int32, (128, 1), 0).astype(jnp.float32)
    qh = jnp.where(qf == iq, p_r, 0.0)
    rh = jnp.where(rf == ir, 1.0, 0.0)
    contrib = lax.dot_general(qh, rh, (((1,), (1,)), ((), ())),
                              preferred_element_type=jnp.float32)

    @pl.when(i == 0)
    def _():
        d_ref[...] = dprev_ref[...]

    d_ref[...] += contrib


def _make_rows(keys, values, idxd, dprev, blk0, nblk, emb_W, emb_b, score_W,
               score_b, query):
    swk = score_W[:KEY_DIM]
    swq = score_W[KEY_DIM:, 0][None, :]
    n_edges = nblk * B_EDGE
    return pl.pallas_call(
        _emb_body,
        grid=(nblk,),
        in_specs=[
            pl.BlockSpec((B_EDGE, KEY_DIM), lambda i: (i + blk0, 0)),
            pl.BlockSpec((B_EDGE, KEY_DIM), lambda i: (i + blk0, 0)),
            pl.BlockSpec((1, 1, B_EDGE), lambda i: (i + blk0, 0, 0)),
            pl.BlockSpec((QDIM, 128), lambda i: (0, 0)),
            pl.BlockSpec((KEY_DIM, OUT_DIM), lambda i: (0, 0)),
            pl.BlockSpec((1, OUT_DIM), lambda i: (0, 0)),
            pl.BlockSpec((KEY_DIM, 1), lambda i: (0, 0)),
            pl.BlockSpec((1, swq.shape[1]), lambda i: (0, 0)),
            pl.BlockSpec((1, swq.shape[1]), lambda i: (0, 0)),
            pl.BlockSpec((1, 1), lambda i: (0, 0)),
        ],
        out_specs=[
            pl.BlockSpec((B_EDGE, OUT_DIM), lambda i: (i, 0)),
            pl.BlockSpec((QDIM, 128), lambda i: (0, 0)),
        ],
        out_shape=[
            jax.ShapeDtypeStruct((n_edges, OUT_DIM), jnp.float32),
            jax.ShapeDtypeStruct((QDIM, 128), jnp.float32),
        ],
        input_output_aliases={3: 1},
    )(keys, values, idxd, dprev, emb_W, emb_b[None, :], swk, swq,
      query[None, :], score_b.reshape(1, 1))


def _sc_zero_init(bufs, acc_s, s):
    zv = jnp.zeros((16,), jnp.float32)

    def zstep(j, carry):
        bufs[0, j // 8, pl.ds((j % 8) * 16, 16)] = zv
        return carry

    lax.fori_loop(0, CH * 8, zstep, 0)
    for k in range(ROWS_PER_SUB // CH):
        pltpu.sync_copy(bufs.at[0],
                        acc_s.at[pl.ds(s * ROWS_PER_SUB + k * CH, CH)])


def _sc_body(n_chunk, e0, first, y_hbm, idx_hbm, init_hbm, out_hbm,
             i0, i1, i2, i3, bufs, acc_s, *sems):
    c = lax.axis_index("c")
    s = lax.axis_index("s")
    wid = c * N_SUB + s
    base = wid * (n_chunk * CH)
    ibase = e0 + base
    idx_bufs = (i0, i1, i2, i3)
    if first:
        _sc_zero_init(bufs, acc_s, s)
    else:
        pltpu.sync_copy(init_hbm.at[c, pl.ds(s * ROWS_PER_SUB, ROWS_PER_SUB)],
                        acc_s.at[pl.ds(s * ROWS_PER_SUB, ROWS_PER_SUB)])
    plsc.subcore_barrier()

    def gather_rows(j, b):
        return pltpu.make_async_copy(
            y_hbm.at[pl.ds(base + j * CH, CH)], bufs.at[b], sems[b])

    def gather_idx(j, b):
        return pltpu.make_async_copy(
            idx_hbm.at[pl.ds(ibase + j * CH, CH)], idx_bufs[b],
            sems[NBUF + b])

    def consume(j, b):
        gather_rows(j, b).wait()
        gather_idx(j, b).wait()
        pltpu.sync_copy(bufs.at[b], acc_s.at[idx_bufs[b]], add=True)

    for b in range(NBUF):
        gather_rows(b, b).start()
        gather_idx(b, b).start()

    def step(jj, carry):
        for b in range(NBUF):
            j = jj * NBUF + b
            consume(j, b)

            @pl.when(j + NBUF < n_chunk)
            def _():
                gather_rows(j + NBUF, b).start()
                gather_idx(j + NBUF, b).start()

        return carry

    lax.fori_loop(0, n_chunk // NBUF, step, 0)
    for b in range(n_chunk % NBUF):
        consume((n_chunk // NBUF) * NBUF + b, b)

    plsc.subcore_barrier()
    pltpu.sync_copy(acc_s.at[pl.ds(s * ROWS_PER_SUB, ROWS_PER_SUB)],
                    out_hbm.at[c, pl.ds(s * ROWS_PER_SUB, ROWS_PER_SUB)])


def _sc_aggregate(rows, idx32, init, e0):
    n_chunk = rows.shape[0] // (N_TILES * CH)
    first = init is None
    if first:
        init = jnp.zeros((1, 1), jnp.float32)
    mesh = plsc.VectorSubcoreMesh(core_axis_name="c", subcore_axis_name="s")
    k = functools.partial(
        pl.kernel,
        mesh=mesh,
        out_type=jax.ShapeDtypeStruct((N_CORES, S_PAD, OUT_DIM), jnp.float32),
        scratch_types=[
            pltpu.VMEM((CH,), jnp.int32),
            pltpu.VMEM((CH,), jnp.int32),
            pltpu.VMEM((CH,), jnp.int32),
            pltpu.VMEM((CH,), jnp.int32),
            pltpu.VMEM((NBUF, CH, OUT_DIM), jnp.float32),
            pltpu.VMEM_SHARED((S_PAD, OUT_DIM), jnp.float32),
        ] + [pltpu.SemaphoreType.DMA] * (2 * NBUF),
    )(functools.partial(_sc_body, n_chunk, e0, first))
    return k(rows, idx32, init)


def _combine_body(pa_ref, den_ref, out_ref):
    a = pa_ref[0] + pa_ref[1]
    out_ref[...] = a[:S] / (den_ref[...] + 1e-16)


def _combine(pa, den_col):
    return pl.pallas_call(
        _combine_body,
        out_shape=jax.ShapeDtypeStruct((S, OUT_DIM), jnp.float32),
    )(pa, den_col)


def kernel(keys, values, query, index, size, emb_W, emb_b, score_W, score_b):
    idx32 = index.astype(jnp.int32)
    idxd = idx32.reshape(N_BLOCKS, 1, B_EDGE)
    dmat = jnp.zeros((QDIM, 128), jnp.float32)
    part = None
    blk0 = 0
    for nblk in SPLITS:
        rows, dmat = _make_rows(keys, values, idxd, dmat, blk0, nblk,
                                emb_W, emb_b, score_W, score_b, query)
        part = _sc_aggregate(rows, idx32, part, blk0 * B_EDGE)
        blk0 += nblk
    den_col = dmat.reshape(-1)[:S, None]
    return _combine(part, den_col)


# speedup vs baseline: 2.1978x; 1.0133x over previous
import functools

import jax
import jax.numpy as jnp
from jax import lax
from jax.experimental import pallas as pl
from jax.experimental.pallas import tpu as pltpu
from jax.experimental.pallas import tpu_sc as plsc

E = 320000
KEY_DIM = 128
OUT_DIM = 128
S = 10000
QDIM = 80
S_PAD = QDIM * 128
B_EDGE = 2560
N_BLOCKS = E // B_EDGE
SPLITS = (34, 34, 34, 23)
N_CORES = 2
N_SUB = 16
N_TILES = N_CORES * N_SUB
CH = 80
ROWS_PER_SUB = S_PAD // N_SUB
NBUF = 4


def _emb_body(keys_ref, vals_ref, idx_ref, dprev_ref, embW_ref, embb_ref,
              swk_ref, swq_ref, q_ref, sb_ref, y_ref, d_ref):
    i = pl.program_id(0)
    c = jnp.sum(q_ref[...] * swq_ref[...]) + sb_ref[0, 0]
    keys = keys_ref[...]
    scores = jnp.dot(keys, swk_ref[...],
                     preferred_element_type=jnp.float32) + c
    p = jnp.exp(scores)
    emb = jnp.dot(vals_ref[...], embW_ref[...],
                  preferred_element_type=jnp.float32) + embb_ref[...]
    y_ref[...] = jnp.maximum(emb, 0.0) * p
    scores_r = lax.dot_general(swk_ref[...].reshape(1, KEY_DIM), keys,
                               (((1,), (1,)), ((), ())),
                               preferred_element_type=jnp.float32) + c
    p_r = jnp.exp(scores_r)
    idx = idx_ref[0]
    qf = (idx // 128).astype(jnp.float32)
    rf = (idx % 128).astype(jnp.float32)
    iq = lax.broadcasted_iota(jnp.int32, (QDIM, 1), 0).astype(jnp.float32)
    ir = lax.broadcasted_iota(jnp.int32, (128, 1), 0).astype(jnp.float32)
    qh = jnp.where(qf == iq, p_r, 0.0)
    rh = jnp.where(rf == ir, 1.0, 0.0)
    contrib = lax.dot_general(qh, rh, (((1,), (1,)), ((), ())),
                              preferred_element_type=jnp.float32)

    @pl.when(i == 0)
    def _():
        d_ref[...] = dprev_ref[...]

    d_ref[...] += contrib


def _make_rows(keys, values, idxd, dprev, blk0, nblk, emb_W, emb_b, score_W,
               score_b, query):
    swk = score_W[:KEY_DIM]
    swq = score_W[KEY_DIM:, 0][None, :]
    n_edges = nblk * B_EDGE
    return pl.pallas_call(
        _emb_body,
        grid=(nblk,),
        in_specs=[
            pl.BlockSpec((B_EDGE, KEY_DIM), lambda i: (i + blk0, 0)),
            pl.BlockSpec((B_EDGE, KEY_DIM), lambda i: (i + blk0, 0)),
            pl.BlockSpec((1, 1, B_EDGE), lambda i: (i + blk0, 0, 0)),
            pl.BlockSpec((QDIM, 128), lambda i: (0, 0)),
            pl.BlockSpec((KEY_DIM, OUT_DIM), lambda i: (0, 0)),
            pl.BlockSpec((1, OUT_DIM), lambda i: (0, 0)),
            pl.BlockSpec((KEY_DIM, 1), lambda i: (0, 0)),
            pl.BlockSpec((1, swq.shape[1]), lambda i: (0, 0)),
            pl.BlockSpec((1, swq.shape[1]), lambda i: (0, 0)),
            pl.BlockSpec((1, 1), lambda i: (0, 0)),
        ],
        out_specs=[
            pl.BlockSpec((B_EDGE, OUT_DIM), lambda i: (i, 0)),
            pl.BlockSpec((QDIM, 128), lambda i: (0, 0)),
        ],
        out_shape=[
            jax.ShapeDtypeStruct((n_edges, OUT_DIM), jnp.float32),
            jax.ShapeDtypeStruct((QDIM, 128), jnp.float32),
        ],
        input_output_aliases={3: 1},
    )(keys, values, idxd, dprev, emb_W, emb_b[None, :], swk, swq,
      query[None, :], score_b.reshape(1, 1))


def _sc_zero_init(bufs, acc_s, s):
    zv = jnp.zeros((16,), jnp.float32)

    def zstep(j, carry):
        bufs[0, j // 8, pl.ds((j % 8) * 16, 16)] = zv
        return carry

    lax.fori_loop(0, CH * 8, zstep, 0)
    for k in range(ROWS_PER_SUB // CH):
        pltpu.sync_copy(bufs.at[0],
                        acc_s.at[pl.ds(s * ROWS_PER_SUB + k * CH, CH)])


def _sc_body(n_chunk, e0, first, y_hbm, idx_hbm, init_hbm, out_hbm,
             i0, i1, i2, i3, bufs, acc_s, *sems):
    c = lax.axis_index("c")
    s = lax.axis_index("s")
    wid = c * N_SUB + s
    base = wid * (n_chunk * CH)
    ibase = e0 + base
    idx_bufs = (i0, i1, i2, i3)
    if first:
        _sc_zero_init(bufs, acc_s, s)
    else:
        pltpu.sync_copy(init_hbm.at[c, pl.ds(s * ROWS_PER_SUB, ROWS_PER_SUB)],
                        acc_s.at[pl.ds(s * ROWS_PER_SUB, ROWS_PER_SUB)])
    plsc.subcore_barrier()

    def gather_rows(j, b):
        return pltpu.make_async_copy(
            y_hbm.at[pl.ds(base + j * CH, CH)], bufs.at[b], sems[b])

    def gather_idx(j, b):
        return pltpu.make_async_copy(
            idx_hbm.at[pl.ds(ibase + j * CH, CH)], idx_bufs[b],
            sems[NBUF + b])

    def consume(j, b):
        gather_rows(j, b).wait()
        gather_idx(j, b).wait()
        pltpu.sync_copy(bufs.at[b], acc_s.at[idx_bufs[b]], add=True)

    for b in range(NBUF):
        gather_rows(b, b).start()
        gather_idx(b, b).start()

    def step(jj, carry):
        for b in range(NBUF):
            j = jj * NBUF + b
            consume(j, b)

            @pl.when(j + NBUF < n_chunk)
            def _():
                gather_rows(j + NBUF, b).start()
                gather_idx(j + NBUF, b).start()

        return carry

    lax.fori_loop(0, n_chunk // NBUF, step, 0)
    for b in range(n_chunk % NBUF):
        consume((n_chunk // NBUF) * NBUF + b, b)

    plsc.subcore_barrier()
    pltpu.sync_copy(acc_s.at[pl.ds(s * ROWS_PER_SUB, ROWS_PER_SUB)],
                    out_hbm.at[c, pl.ds(s * ROWS_PER_SUB, ROWS_PER_SUB)])


def _sc_aggregate(rows, idx32, init, e0):
    n_chunk = rows.shape[0] // (N_TILES * CH)
    first = init is None
    if first:
        init = jnp.zeros((1, 1), jnp.float32)
    mesh = plsc.VectorSubcoreMesh(core_axis_name="c", subcore_axis_name="s")
    k = functools.partial(
        pl.kernel,
        mesh=mesh,
        out_type=jax.ShapeDtypeStruct((N_CORES, S_PAD, OUT_DIM), jnp.float32),
        scratch_types=[
            pltpu.VMEM((CH,), jnp.int32),
            pltpu.VMEM((CH,), jnp.int32),
            pltpu.VMEM((CH,), jnp.int32),
            pltpu.VMEM((CH,), jnp.int32),
            pltpu.VMEM((NBUF, CH, OUT_DIM), jnp.float32),
            pltpu.VMEM_SHARED((S_PAD, OUT_DIM), jnp.float32),
        ] + [pltpu.SemaphoreType.DMA] * (2 * NBUF),
    )(functools.partial(_sc_body, n_chunk, e0, first))
    return k(rows, idx32, init)


def _combine_body(pa_ref, den_ref, out_ref):
    a = pa_ref[0] + pa_ref[1]
    out_ref[...] = a[:S] / (den_ref[...] + 1e-16)


def _combine(pa, den_col):
    return pl.pallas_call(
        _combine_body,
        out_shape=jax.ShapeDtypeStruct((S, OUT_DIM), jnp.float32),
    )(pa, den_col)


def kernel(keys, values, query, index, size, emb_W, emb_b, score_W, score_b):
    idx32 = index.astype(jnp.int32)
    idxd = idx32.reshape(N_BLOCKS, 1, B_EDGE)
    dmat = jnp.zeros((QDIM, 128), jnp.float32)
    part = None
    blk0 = 0
    for nblk in SPLITS:
        rows, dmat = _make_rows(keys, values, idxd, dmat, blk0, nblk,
                                emb_W, emb_b, score_W, score_b, query)
        part = _sc_aggregate(rows, idx32, part, blk0 * B_EDGE)
        blk0 += nblk
    den_col = dmat.reshape(-1)[:S, None]
    return _combine(part, den_col)
